# Initial kernel scaffold; baseline (speedup 1.0000x reference)
import jax
import jax.numpy as jnp
from jax.experimental import pallas as pl


def kernel(x_user, x_item, edge_index_user_item, edge_index_item_user, basis0, comp0, root0, bias0, basis1, comp1, root1, bias1):
    raise NotImplementedError("write your pallas kernel here")


# speedup vs baseline: 2.1461x; 2.1461x over previous
import functools

import jax
import jax.numpy as jnp
from jax import lax
from jax.experimental import pallas as pl
from jax.experimental.pallas import tpu as pltpu
from jax.experimental.pallas import tpu_sc as plsc

N = 10000
NP = 10240
D = 128
DH = 64
E = 160000
NSUB = 16
K = 80
EPT = 10240
CH = EPT // K
RPT = NP // NSUB
ZR = 128
DEGW = 16
NB = 4


def _sc_agg_body(with_deg, x_hbm, src_hbm, dst_hbm, out_hbm, deg_hbm,
                 srcb, dstb, rows, ones, zbuf, degbuf, acc, dega, sem):
  c = lax.axis_index("c")
  s = lax.axis_index("s")

  z16 = jnp.zeros((16,), jnp.float32)

  def _zrow(r, _):
    def _zcol(k8, _):
      zbuf[r, pl.ds(k8 * 16, 16)] = z16
      return 0
    lax.fori_loop(0, DH // 16, _zcol, 0)
    degbuf[r, :] = z16
    return 0
  lax.fori_loop(0, ZR, _zrow, 0)

  if with_deg:
    o16 = jnp.ones((16,), jnp.float32)
    def _orow(r, _):
      ones[r, :] = o16
      return 0
    lax.fori_loop(0, K, _orow, 0)

  def _zacc(k, _):
    pltpu.sync_copy(zbuf, acc.at[pl.ds(s * RPT + k * ZR, ZR)])
    if with_deg:
      pltpu.sync_copy(degbuf, dega.at[pl.ds(s * RPT + k * ZR, ZR)])
    return 0
  lax.fori_loop(0, RPT // ZR, _zacc, 0)

  plsc.subcore_barrier()

  row0 = (c * NSUB + s) * CH
  pltpu.sync_copy(src_hbm.at[pl.ds(row0, CH)], srcb)
  pltpu.sync_copy(dst_hbm.at[pl.ds(row0, CH)], dstb)

  def _chunk(j, _):
    pltpu.async_copy(x_hbm.at[srcb.at[j]], rows, sem).wait()
    pltpu.sync_copy(rows, acc.at[dstb.at[j]], add=True)
    if with_deg:
      pltpu.sync_copy(ones, dega.at[dstb.at[j]], add=True)
    return 0
  lax.fori_loop(0, CH, _chunk, 0)

  plsc.subcore_barrier()

  def _wout(k, _):
    r = s * RPT + k * ZR
    pltpu.sync_copy(acc.at[pl.ds(r, ZR)], zbuf)
    pltpu.sync_copy(zbuf, out_hbm.at[pl.ds(c * NP + r, ZR)])
    if with_deg:
      pltpu.sync_copy(dega.at[pl.ds(r, ZR)], degbuf)
      pltpu.sync_copy(degbuf, deg_hbm.at[pl.ds(c * NP + r, ZR)])
    return 0
  lax.fori_loop(0, RPT // ZR, _wout, 0)


def _make_sc_agg(with_deg):
  mesh = plsc.VectorSubcoreMesh(core_axis_name="c", subcore_axis_name="s")
  out_type = [jax.ShapeDtypeStruct((2 * NP, DH), jnp.float32),
              jax.ShapeDtypeStruct((2 * NP, DEGW), jnp.float32)]
  scratch = [
      pltpu.VMEM((CH, K), jnp.int32),
      pltpu.VMEM((CH, K), jnp.int32),
      pltpu.VMEM((K, DH), jnp.float32),
      pltpu.VMEM((K, DEGW), jnp.float32),
      pltpu.VMEM((ZR, DH), jnp.float32),
      pltpu.VMEM((ZR, DEGW), jnp.float32),
      pltpu.VMEM_SHARED((NP, DH), jnp.float32),
      pltpu.VMEM_SHARED((NP, DEGW), jnp.float32),
      pltpu.SemaphoreType.DMA,
  ]
  return pl.kernel(functools.partial(_sc_agg_body, with_deg),
                   out_type=out_type, mesh=mesh, scratch_types=scratch,
                   compiler_params=pltpu.CompilerParams(
                       use_tc_tiling_on_sc=False))


_sc_agg_deg = _make_sc_agg(True)
_sc_agg = _make_sc_agg(False)

BR = 1280


def _tc_dense_body(relu, xlo_ref, xhi_ref, alo_ref, ahi_ref, deg_ref,
                   basis_ref, comp_ref, root_ref, bias_ref,
                   olo_ref, ohi_ref):
  g = pl.program_id(0)
  wu = jnp.zeros((D, D), jnp.float32)
  wi = jnp.zeros((D, D), jnp.float32)
  for b in range(NB):
    wu = wu + comp_ref[1, b] * basis_ref[b]
    wi = wi + comp_ref[0, b] * basis_ref[b]
  w = jnp.where(g < (NP // BR), wu, wi)
  dinv = 1.0 / jnp.maximum(deg_ref[:, 0:1], 1.0)
  root = root_ref[...]
  h = (jnp.dot(xlo_ref[...], root[:DH], preferred_element_type=jnp.float32)
       + jnp.dot(xhi_ref[...], root[DH:], preferred_element_type=jnp.float32)
       + jnp.dot(alo_ref[...] * dinv, w[:DH],
                 preferred_element_type=jnp.float32)
       + jnp.dot(ahi_ref[...] * dinv, w[DH:],
                 preferred_element_type=jnp.float32)
       + bias_ref[...])
  if relu:
    h = jnp.maximum(h, 0.0)
  olo_ref[...] = h[:, :DH]
  ohi_ref[...] = h[:, DH:]


def _make_tc_dense(relu):
  half = pl.BlockSpec((BR, DH), lambda g: (g, 0))
  return pl.pallas_call(
      functools.partial(_tc_dense_body, relu),
      grid=(2 * NP // BR,),
      in_specs=[
          half, half, half, half,
          pl.BlockSpec((BR, DEGW), lambda g: (g, 0)),
          pl.BlockSpec((NB, D, D), lambda g: (0, 0, 0)),
          pl.BlockSpec(memory_space=pltpu.SMEM),
          pl.BlockSpec((D, D), lambda g: (0, 0)),
          pl.BlockSpec((1, D), lambda g: (0, 0)),
      ],
      out_specs=[half, half],
      out_shape=[jax.ShapeDtypeStruct((2 * NP, DH), jnp.float32),
                 jax.ShapeDtypeStruct((2 * NP, DH), jnp.float32)],
  )


_tc_dense_relu = _make_tc_dense(True)
_tc_dense_lin = _make_tc_dense(False)


def _prep_edges(ei, src_off):
  src = (ei[0].astype(jnp.int32) + src_off).reshape(NSUB, E // NSUB)
  dst = ei[1].astype(jnp.int32).reshape(NSUB, E // NSUB)
  pad = EPT - E // NSUB
  src = jnp.pad(src, ((0, 0), (0, pad)), constant_values=0)
  dst = jnp.pad(dst, ((0, 0), (0, pad)), constant_values=N)
  return src, dst


def _pad_half(xu, xi, col):
  zpad = jnp.zeros((NP - N, DH), jnp.float32)
  return jnp.concatenate([xu[:, col:col + DH], zpad,
                          xi[:, col:col + DH], zpad], axis=0)


def kernel(x_user, x_item, edge_index_user_item, edge_index_item_user,
           basis0, comp0, root0, bias0, basis1, comp1, root1, bias1):
  s0, d0 = _prep_edges(edge_index_item_user, NP)
  s1, d1 = _prep_edges(edge_index_user_item, 0)
  src2 = jnp.stack([s0, s1]).reshape(2 * NSUB * CH, K)
  dst2 = jnp.stack([d0, d1]).reshape(2 * NSUB * CH, K)

  x0lo = _pad_half(x_user, x_item, 0)
  x0hi = _pad_half(x_user, x_item, DH)

  a0lo, deg = _sc_agg_deg(x0lo, src2, dst2)
  a0hi, _ = _sc_agg(x0hi, src2, dst2)
  h1lo, h1hi = _tc_dense_relu(x0lo, x0hi, a0lo, a0hi, deg, basis0, comp0,
                              root0, bias0.reshape(1, D))
  a1lo, _ = _sc_agg(h1lo, src2, dst2)
  a1hi, _ = _sc_agg(h1hi, src2, dst2)
  h2lo, h2hi = _tc_dense_lin(h1lo, h1hi, a1lo, a1hi, deg, basis1, comp1,
                             root1, bias1.reshape(1, D))
  out = jnp.concatenate([h2lo, h2hi], axis=1)
  return (out[:N], out[NP:NP + N])


# speedup vs baseline: 2.9506x; 1.3748x over previous
import functools

import jax
import jax.numpy as jnp
from jax import lax
from jax.experimental import pallas as pl
from jax.experimental.pallas import tpu as pltpu
from jax.experimental.pallas import tpu_sc as plsc

N = 10000
NP = 10240
D = 128
DH = 64
E = 160000
NSUB = 16
K = 128
EPT = 10240
CH = EPT // K
PD = 4
RPT = NP // NSUB
ZR = 128
DEGW = 16
NB = 4


def _sc_agg_body(with_deg, x_hbm, src_hbm, dst_hbm, out_hbm, deg_hbm,
                 srcb, dstb, rows, ones, zbuf, degbuf, acc, dega, sems):
  c = lax.axis_index("c")
  s = lax.axis_index("s")

  z16 = jnp.zeros((16,), jnp.float32)

  def _zrow(r, _):
    def _zcol(k8, _):
      zbuf[r, pl.ds(k8 * 16, 16)] = z16
      return 0
    lax.fori_loop(0, DH // 16, _zcol, 0)
    degbuf[r, :] = z16
    return 0
  lax.fori_loop(0, ZR, _zrow, 0)

  if with_deg:
    o16 = jnp.ones((16,), jnp.float32)
    def _orow(r, _):
      ones[r, :] = o16
      return 0
    lax.fori_loop(0, K, _orow, 0)

  def _zacc(k, _):
    pltpu.sync_copy(zbuf, acc.at[pl.ds(s * RPT + k * ZR, ZR)])
    if with_deg:
      pltpu.sync_copy(degbuf, dega.at[pl.ds(s * RPT + k * ZR, ZR)])
    return 0
  lax.fori_loop(0, RPT // ZR, _zacc, 0)

  plsc.subcore_barrier()

  row0 = (c * NSUB + s) * CH
  pltpu.sync_copy(src_hbm.at[pl.ds(row0, CH)], srcb)
  pltpu.sync_copy(dst_hbm.at[pl.ds(row0, CH)], dstb)

  def _fire(j, b):
    pltpu.async_copy(x_hbm.at[srcb.at[j]], rows[b], sems[b])

  for b in range(PD):
    _fire(b, b)

  def _group(g, _):
    for b in range(PD):
      j = g * PD + b
      pltpu.make_async_copy(x_hbm.at[srcb.at[j]], rows[b], sems[b]).wait()
      pltpu.sync_copy(rows[b], acc.at[dstb.at[j]], add=True)
      if with_deg:
        pltpu.sync_copy(ones, dega.at[dstb.at[j]], add=True)
      @pl.when(j + PD < CH)
      def _():
        _fire(j + PD, b)
    return 0
  lax.fori_loop(0, CH // PD, _group, 0)

  plsc.subcore_barrier()

  def _wout(k, _):
    r = s * RPT + k * ZR
    pltpu.sync_copy(acc.at[pl.ds(r, ZR)], zbuf)
    pltpu.sync_copy(zbuf, out_hbm.at[pl.ds(c * NP + r, ZR)])
    if with_deg:
      pltpu.sync_copy(dega.at[pl.ds(r, ZR)], degbuf)
      pltpu.sync_copy(degbuf, deg_hbm.at[pl.ds(c * NP + r, ZR)])
    return 0
  lax.fori_loop(0, RPT // ZR, _wout, 0)


def _make_sc_agg(with_deg):
  mesh = plsc.VectorSubcoreMesh(core_axis_name="c", subcore_axis_name="s")
  out_type = [jax.ShapeDtypeStruct((2 * NP, DH), jnp.float32),
              jax.ShapeDtypeStruct((2 * NP, DEGW), jnp.float32)]
  scratch = [
      pltpu.VMEM((CH, K), jnp.int32),
      pltpu.VMEM((CH, K), jnp.int32),
      [pltpu.VMEM((K, DH), jnp.float32) for _ in range(PD)],
      pltpu.VMEM((K, DEGW), jnp.float32),
      pltpu.VMEM((ZR, DH), jnp.float32),
      pltpu.VMEM((ZR, DEGW), jnp.float32),
      pltpu.VMEM_SHARED((NP, DH), jnp.float32),
      pltpu.VMEM_SHARED((NP, DEGW), jnp.float32),
      [pltpu.SemaphoreType.DMA for _ in range(PD)],
  ]
  return pl.kernel(functools.partial(_sc_agg_body, with_deg),
                   out_type=out_type, mesh=mesh, scratch_types=scratch,
                   compiler_params=pltpu.CompilerParams(
                       use_tc_tiling_on_sc=False))


_sc_agg_deg = _make_sc_agg(True)
_sc_agg = _make_sc_agg(False)

BR = 1280


def _tc_dense_body(relu, xlo_ref, xhi_ref, alo_ref, ahi_ref, deg_ref,
                   basis_ref, comp_ref, root_ref, bias_ref,
                   olo_ref, ohi_ref):
  g = pl.program_id(0)
  wu = jnp.zeros((D, D), jnp.float32)
  wi = jnp.zeros((D, D), jnp.float32)
  for b in range(NB):
    wu = wu + comp_ref[1, b] * basis_ref[b]
    wi = wi + comp_ref[0, b] * basis_ref[b]
  w = jnp.where(g < (NP // BR), wu, wi)
  dinv = 1.0 / jnp.maximum(deg_ref[:, 0:1], 1.0)
  root = root_ref[...]
  h = (jnp.dot(xlo_ref[...], root[:DH], preferred_element_type=jnp.float32)
       + jnp.dot(xhi_ref[...], root[DH:], preferred_element_type=jnp.float32)
       + jnp.dot(alo_ref[...] * dinv, w[:DH],
                 preferred_element_type=jnp.float32)
       + jnp.dot(ahi_ref[...] * dinv, w[DH:],
                 preferred_element_type=jnp.float32)
       + bias_ref[...])
  if relu:
    h = jnp.maximum(h, 0.0)
  olo_ref[...] = h[:, :DH]
  ohi_ref[...] = h[:, DH:]


def _make_tc_dense(relu):
  half = pl.BlockSpec((BR, DH), lambda g: (g, 0))
  return pl.pallas_call(
      functools.partial(_tc_dense_body, relu),
      grid=(2 * NP // BR,),
      in_specs=[
          half, half, half, half,
          pl.BlockSpec((BR, DEGW), lambda g: (g, 0)),
          pl.BlockSpec((NB, D, D), lambda g: (0, 0, 0)),
          pl.BlockSpec(memory_space=pltpu.SMEM),
          pl.BlockSpec((D, D), lambda g: (0, 0)),
          pl.BlockSpec((1, D), lambda g: (0, 0)),
      ],
      out_specs=[half, half],
      out_shape=[jax.ShapeDtypeStruct((2 * NP, DH), jnp.float32),
                 jax.ShapeDtypeStruct((2 * NP, DH), jnp.float32)],
  )


_tc_dense_relu = _make_tc_dense(True)
_tc_dense_lin = _make_tc_dense(False)


def _prep_edges(ei, src_off):
  src = (ei[0].astype(jnp.int32) + src_off).reshape(NSUB, E // NSUB)
  dst = ei[1].astype(jnp.int32).reshape(NSUB, E // NSUB)
  pad = EPT - E // NSUB
  src = jnp.pad(src, ((0, 0), (0, pad)), constant_values=0)
  dst = jnp.pad(dst, ((0, 0), (0, pad)), constant_values=N)
  return src, dst


def _pad_half(xu, xi, col):
  zpad = jnp.zeros((NP - N, DH), jnp.float32)
  return jnp.concatenate([xu[:, col:col + DH], zpad,
                          xi[:, col:col + DH], zpad], axis=0)


def kernel(x_user, x_item, edge_index_user_item, edge_index_item_user,
           basis0, comp0, root0, bias0, basis1, comp1, root1, bias1):
  s0, d0 = _prep_edges(edge_index_item_user, NP)
  s1, d1 = _prep_edges(edge_index_user_item, 0)
  src2 = jnp.stack([s0, s1]).reshape(2 * NSUB * CH, K)
  dst2 = jnp.stack([d0, d1]).reshape(2 * NSUB * CH, K)

  x0lo = _pad_half(x_user, x_item, 0)
  x0hi = _pad_half(x_user, x_item, DH)

  a0lo, deg = _sc_agg_deg(x0lo, src2, dst2)
  a0hi, _ = _sc_agg(x0hi, src2, dst2)
  h1lo, h1hi = _tc_dense_relu(x0lo, x0hi, a0lo, a0hi, deg, basis0, comp0,
                              root0, bias0.reshape(1, D))
  a1lo, _ = _sc_agg(h1lo, src2, dst2)
  a1hi, _ = _sc_agg(h1hi, src2, dst2)
  h2lo, h2hi = _tc_dense_lin(h1lo, h1hi, a1lo, a1hi, deg, basis1, comp1,
                             root1, bias1.reshape(1, D))
  out = jnp.concatenate([h2lo, h2hi], axis=1)
  return (out[:N], out[NP:NP + N])
